# Initial kernel scaffold; baseline (speedup 1.0000x reference)
import jax
import jax.numpy as jnp
from jax.experimental import pallas as pl


def kernel(word_ids, age_ids, seg_ids, posi_ids, word_table, seg_table, age_table, posi_table, gamma, beta):
    raise NotImplementedError("write your pallas kernel here")


# speedup vs baseline: 9.5332x; 9.5332x over previous
import functools

import jax
import jax.numpy as jnp
from jax import lax
from jax.experimental import pallas as pl
from jax.experimental.pallas import tpu as pltpu
from jax.experimental.pallas import tpu_sc as plsc

H = 64
NC, NS = 2, 16
NW = NC * NS
G = 128
C = 512
SUB = C // G
UNROLL = 8


def _rsqrt(v):
    i = lax.bitcast_convert_type(v, jnp.int32)
    i = jnp.int32(0x5F3759DF) - lax.shift_right_logical(i, 1)
    y = lax.bitcast_convert_type(i, jnp.float32)
    for _ in range(3):
        y = y * (jnp.float32(1.5) - jnp.float32(0.5) * v * y * y)
    return y


def _sc_body(wid_hbm, fid_hbm, wtab, ftab, gamma_in, beta_in, out_hbm,
             idx_w, idx_f, buf_w, buf_f, gam_v, bet_v, sem):
    w = lax.axis_index("s") * NC + lax.axis_index("c")
    n_tok = out_hbm.shape[0]
    per_w = n_tok // NW
    n_chunks = per_w // C
    rows_w = per_w // G

    pltpu.sync_copy(wid_hbm.at[pl.ds(w * rows_w, rows_w)], idx_w)
    pltpu.sync_copy(fid_hbm.at[pl.ds(w * rows_w, rows_w)], idx_f)
    pltpu.sync_copy(gamma_in, gam_v)
    pltpu.sync_copy(beta_in, bet_v)
    gvec = [gam_v[pl.ds(16 * k, 16)] for k in range(4)]
    bvec = [bet_v[pl.ds(16 * k, 16)] for k in range(4)]

    lanes = lax.iota(jnp.int32, 16)
    perms = [lanes ^ st for st in (8, 4, 2, 1)]

    def allsum(v):
        for p in perms:
            v = v + v.at[p].get(mode="promise_in_bounds")
        return v

    def chunk_body(c, _):
        tok0 = w * per_w + c * C
        descs = []
        for j in range(SUB):
            descs.append(pltpu.async_copy(
                wtab.at[idx_w.at[c * SUB + j]], buf_w.at[pl.ds(j * G, G)],
                sem))
            descs.append(pltpu.async_copy(
                ftab.at[idx_f.at[c * SUB + j]], buf_f.at[pl.ds(j * G, G)],
                sem))
        for d in descs:
            d.wait()

        def tok_body(i, _):
            for uu in range(UNROLL):
                t = i * UNROLL + uu
                x = [buf_w[t, pl.ds(16 * k, 16)] + buf_f[t, pl.ds(16 * k, 16)]
                     for k in range(4)]
                s = allsum((x[0] + x[1]) + (x[2] + x[3]))
                q = allsum((x[0] * x[0] + x[1] * x[1])
                           + (x[2] * x[2] + x[3] * x[3]))
                u = s * jnp.float32(1.0 / H)
                var = q * jnp.float32(1.0 / H) - u * u
                r = _rsqrt(var + jnp.float32(1e-12))
                for k in range(4):
                    buf_w[t, pl.ds(16 * k, 16)] = (
                        (x[k] - u) * r * gvec[k] + bvec[k])
            return 0

        lax.fori_loop(0, C // UNROLL, tok_body, 0)
        pltpu.sync_copy(buf_w, out_hbm.at[pl.ds(tok0, C)])
        return 0

    lax.fori_loop(0, n_chunks, chunk_body, 0)


@jax.jit
def _sc_embed(wid2d, fid2d, wtab, ftab, gamma, beta):
    n_tok = wid2d.shape[0] * wid2d.shape[1]
    mesh = plsc.VectorSubcoreMesh(core_axis_name="c", subcore_axis_name="s")
    return pl.kernel(
        _sc_body,
        out_type=jax.ShapeDtypeStruct((n_tok, H), jnp.float32),
        mesh=mesh,
        scratch_types=[
            pltpu.VMEM((n_tok // NW // G, G), jnp.int32),
            pltpu.VMEM((n_tok // NW // G, G), jnp.int32),
            pltpu.VMEM((C, H), jnp.float32),
            pltpu.VMEM((C, H), jnp.float32),
            pltpu.VMEM((H,), jnp.float32),
            pltpu.VMEM((H,), jnp.float32),
            pltpu.SemaphoreType.DMA,
        ],
        compiler_params=pltpu.CompilerParams(use_tc_tiling_on_sc=False),
    )(wid2d, fid2d, wtab, ftab, gamma, beta)


def kernel(word_ids, age_ids, seg_ids, posi_ids, word_table, seg_table,
           age_table, posi_table, gamma, beta):
    B, L = word_ids.shape
    n_tok = B * L
    segv, h = seg_table.shape
    agev = age_table.shape[0]
    posv = posi_table.shape[0]
    ftab = (seg_table[:, None, None, :] + age_table[None, :, None, :]
            + posi_table[None, None, :, :]).reshape(segv * agev * posv, h)
    wid = word_ids.reshape(n_tok).astype(jnp.int32)
    fid = ((seg_ids.astype(jnp.int32) * agev + age_ids.astype(jnp.int32))
           * posv + posi_ids.astype(jnp.int32)).reshape(n_tok)
    out = _sc_embed(wid.reshape(n_tok // G, G), fid.reshape(n_tok // G, G),
                    word_table, ftab, gamma, beta)
    return out.reshape(B, L, h)
